# baseline (speedup 1.0000x reference)
import functools

import jax
import jax.numpy as jnp
from jax import lax
from jax.experimental import pallas as pl
from jax.experimental.pallas import tpu as pltpu
from jax.experimental.pallas import tpu_sc as plsc

N_NODES = 10000
N_EDGES = 160000
D_FEAT = 256
D_HALF = 128
N_CORES = 2
N_TILES = 16

BLOCK_E = 128
N_BLOCKS = 80
EDGES_PER_TILE = N_BLOCKS * BLOCK_E
E_PAD = EDGES_PER_TILE * N_TILES
N_PAD = 10240
ROWS_PER_TILE = N_PAD // N_TILES


def _weight_body(x_ref, cj_ref, w0_ref, w1_ref):
    w0_ref[...] = x_ref[:, :D_HALF] * cj_ref[...]
    w1_ref[...] = x_ref[:, D_HALF:] * cj_ref[...]


_R = 1000

_tc_weight = pl.pallas_call(
    _weight_body,
    grid=(N_NODES // _R,),
    in_specs=[
        pl.BlockSpec((_R, D_FEAT), lambda i: (i, 0)),
        pl.BlockSpec((_R, 1), lambda i: (i, 0)),
    ],
    out_specs=[
        pl.BlockSpec((_R, D_HALF), lambda i: (i, 0)),
        pl.BlockSpec((_R, D_HALF), lambda i: (i, 0)),
    ],
    out_shape=[
        jax.ShapeDtypeStruct((N_NODES, D_HALF), jnp.float32),
        jax.ShapeDtypeStruct((N_NODES, D_HALF), jnp.float32),
    ],
)


def _sc_body(w0_hbm, w1_hbm, idx_hbm, zero_hbm, out_hbm,
             ib0, ib1, rows0, rows1, acc,
             isem0, isem1, gsem0, gsem1, ssem0, ssem1):
    c = lax.axis_index("c")
    s = lax.axis_index("s")
    out_off = c * N_PAD

    def idx_start(i, ib, sem):
        pltpu.async_copy(idx_hbm.at[s, i], ib, sem)

    def idx_wait(ib, sem):
        pltpu.make_async_copy(idx_hbm.at[0, 0], ib, sem).wait()

    def start_gather(ib, rows, sem):
        @pl.when(c == 0)
        def _():
            pltpu.async_copy(w0_hbm.at[ib.at[0]], rows, sem)

        @pl.when(c == 1)
        def _():
            pltpu.async_copy(w1_hbm.at[ib.at[0]], rows, sem)

    def wait_gather(rows, sem):
        pltpu.make_async_copy(w0_hbm.at[pl.ds(0, BLOCK_E)], rows, sem).wait()

    def start_scatter(ib, rows, sem):
        pltpu.async_copy(rows, acc.at[ib.at[1]], sem, add=True)

    def wait_scatter(rows, sem):
        pltpu.make_async_copy(rows, acc.at[pl.ds(0, BLOCK_E)], sem).wait()

    idx_start(0, ib0, isem0)
    r0 = s * ROWS_PER_TILE
    pltpu.sync_copy(zero_hbm.at[pl.ds(r0, ROWS_PER_TILE)],
                    acc.at[pl.ds(r0, ROWS_PER_TILE)])
    plsc.subcore_barrier()

    idx_wait(ib0, isem0)
    start_gather(ib0, rows0, gsem0)

    def block2(k, carry):
        e = 2 * k
        wait_gather(rows0, gsem0)
        start_scatter(ib0, rows0, ssem0)

        @pl.when(k > 0)
        def _():
            wait_scatter(rows1, ssem1)

        idx_start(e + 1, ib1, isem1)
        idx_wait(ib1, isem1)
        start_gather(ib1, rows1, gsem1)
        wait_scatter(rows0, ssem0)

        @pl.when(e + 2 < N_BLOCKS)
        def _():
            idx_start(e + 2, ib0, isem0)
            idx_wait(ib0, isem0)

        wait_gather(rows1, gsem1)
        start_scatter(ib1, rows1, ssem1)

        @pl.when(e + 2 < N_BLOCKS)
        def _():
            start_gather(ib0, rows0, gsem0)

        return carry

    lax.fori_loop(0, N_BLOCKS // 2, block2, 0)

    wait_scatter(rows1, ssem1)
    plsc.subcore_barrier()
    pltpu.sync_copy(acc.at[pl.ds(r0, ROWS_PER_TILE)],
                    out_hbm.at[pl.ds(out_off + r0, ROWS_PER_TILE)])


_sc_gather_scatter = functools.partial(
    pl.kernel,
    out_type=jax.ShapeDtypeStruct((N_CORES * N_PAD, D_HALF), jnp.float32),
    mesh=plsc.VectorSubcoreMesh(core_axis_name="c", subcore_axis_name="s"),
    scratch_types=[
        pltpu.VMEM((2, BLOCK_E), jnp.int32),
        pltpu.VMEM((2, BLOCK_E), jnp.int32),
        pltpu.VMEM((BLOCK_E, D_HALF), jnp.float32),
        pltpu.VMEM((BLOCK_E, D_HALF), jnp.float32),
        pltpu.VMEM_SHARED((N_PAD, D_HALF), jnp.float32),
        pltpu.SemaphoreType.DMA,
        pltpu.SemaphoreType.DMA,
        pltpu.SemaphoreType.DMA,
        pltpu.SemaphoreType.DMA,
        pltpu.SemaphoreType.DMA,
        pltpu.SemaphoreType.DMA,
    ],
)(_sc_body)


def _scale_body(a_ref, ci_ref, o_ref):
    o_ref[:, :D_HALF] = a_ref[0] * ci_ref[...]
    o_ref[:, D_HALF:] = a_ref[1] * ci_ref[...]


_tc_scale = pl.pallas_call(
    _scale_body,
    grid=(N_NODES // _R,),
    in_specs=[
        pl.BlockSpec((N_CORES, _R, D_HALF), lambda i: (0, i, 0)),
        pl.BlockSpec((_R, 1), lambda i: (i, 0)),
    ],
    out_specs=pl.BlockSpec((_R, D_FEAT), lambda i: (i, 0)),
    out_shape=jax.ShapeDtypeStruct((N_NODES, D_FEAT), jnp.float32),
)


def kernel(src_feats, edge_index, cj, ci):
    src = edge_index[0].astype(jnp.int32)
    dst = edge_index[1].astype(jnp.int32)
    pad = E_PAD - N_EDGES
    src3 = jnp.concatenate([src, jnp.zeros((pad,), jnp.int32)])
    src3 = src3.reshape(N_TILES, N_BLOCKS, BLOCK_E)
    dst3 = jnp.concatenate([dst, jnp.full((pad,), N_NODES, jnp.int32)])
    dst3 = dst3.reshape(N_TILES, N_BLOCKS, BLOCK_E)
    idx = jnp.stack([src3, dst3], axis=2)

    w0, w1 = _tc_weight(src_feats, cj)
    zeros = jnp.zeros((N_PAD, D_HALF), jnp.float32)
    agg = _sc_gather_scatter(w0, w1, idx, zeros)
    return _tc_scale(agg.reshape(N_CORES, N_PAD, D_HALF), ci)


# speedup vs baseline: 1.0027x; 1.0027x over previous
import functools

import jax
import jax.numpy as jnp
from jax import lax
from jax.experimental import pallas as pl
from jax.experimental.pallas import tpu as pltpu
from jax.experimental.pallas import tpu_sc as plsc

N_NODES = 10000
N_EDGES = 160000
D_FEAT = 256
D_HALF = 128
N_CORES = 2
N_TILES = 16

BLOCK_E = 128
N_BLOCKS = 80
EDGES_PER_TILE = N_BLOCKS * BLOCK_E
E_PAD = EDGES_PER_TILE * N_TILES
N_PAD = 10240
ROWS_PER_TILE = N_PAD // N_TILES


def _weight_body(x_ref, cj_ref, w0_ref, w1_ref):
    w0_ref[...] = x_ref[:, :D_HALF] * cj_ref[...]
    w1_ref[...] = x_ref[:, D_HALF:] * cj_ref[...]


_R = 1000

_tc_weight = pl.pallas_call(
    _weight_body,
    grid=(N_NODES // _R,),
    in_specs=[
        pl.BlockSpec((_R, D_FEAT), lambda i: (i, 0)),
        pl.BlockSpec((_R, 1), lambda i: (i, 0)),
    ],
    out_specs=[
        pl.BlockSpec((_R, D_HALF), lambda i: (i, 0)),
        pl.BlockSpec((_R, D_HALF), lambda i: (i, 0)),
    ],
    out_shape=[
        jax.ShapeDtypeStruct((N_NODES, D_HALF), jnp.float32),
        jax.ShapeDtypeStruct((N_NODES, D_HALF), jnp.float32),
    ],
)


def _sc_body(w0_hbm, w1_hbm, idx_hbm, zero_hbm, out_hbm,
             ib0, ib1, rows0, rows1, acc,
             isem0, isem1, gsem0, gsem1, ssem0, ssem1):
    c = lax.axis_index("c")
    s = lax.axis_index("s")
    out_off = c * N_PAD

    def idx_start(i, ib, sem):
        pltpu.async_copy(idx_hbm.at[s, i], ib, sem)

    def idx_wait(ib, sem):
        pltpu.make_async_copy(idx_hbm.at[0, 0], ib, sem).wait()

    def start_gather(ib, rows, sem):
        @pl.when(c == 0)
        def _():
            pltpu.async_copy(w0_hbm.at[ib.at[0]], rows, sem)

        @pl.when(c == 1)
        def _():
            pltpu.async_copy(w1_hbm.at[ib.at[0]], rows, sem)

    def wait_gather(rows, sem):
        pltpu.make_async_copy(w0_hbm.at[pl.ds(0, BLOCK_E)], rows, sem).wait()

    def start_scatter(ib, rows, sem):
        pltpu.async_copy(rows, acc.at[ib.at[1]], sem, add=True)

    def wait_scatter(rows, sem):
        pltpu.make_async_copy(rows, acc.at[pl.ds(0, BLOCK_E)], sem).wait()

    idx_start(0, ib0, isem0)
    r0 = s * ROWS_PER_TILE
    pltpu.sync_copy(zero_hbm.at[pl.ds(r0, ROWS_PER_TILE)],
                    acc.at[pl.ds(r0, ROWS_PER_TILE)])
    plsc.subcore_barrier()

    idx_wait(ib0, isem0)
    start_gather(ib0, rows0, gsem0)

    def block2(k, carry):
        e = 2 * k
        wait_gather(rows0, gsem0)
        start_scatter(ib0, rows0, ssem0)

        @pl.when(k > 0)
        def _():
            wait_scatter(rows1, ssem1)

        idx_start(e + 1, ib1, isem1)
        idx_wait(ib1, isem1)
        start_gather(ib1, rows1, gsem1)
        wait_scatter(rows0, ssem0)

        @pl.when(e + 2 < N_BLOCKS)
        def _():
            idx_start(e + 2, ib0, isem0)
            idx_wait(ib0, isem0)

        wait_gather(rows1, gsem1)
        start_scatter(ib1, rows1, ssem1)

        @pl.when(e + 2 < N_BLOCKS)
        def _():
            start_gather(ib0, rows0, gsem0)

        return carry

    lax.fori_loop(0, N_BLOCKS // 2, block2, 0)

    wait_scatter(rows1, ssem1)
    plsc.subcore_barrier()
    pltpu.sync_copy(acc.at[pl.ds(r0, ROWS_PER_TILE)],
                    out_hbm.at[pl.ds(out_off + r0, ROWS_PER_TILE)])


_sc_gather_scatter = functools.partial(
    pl.kernel,
    out_type=jax.ShapeDtypeStruct((N_CORES * N_PAD, D_HALF), jnp.float32),
    mesh=plsc.VectorSubcoreMesh(core_axis_name="c", subcore_axis_name="s"),
    scratch_types=[
        pltpu.VMEM((2, BLOCK_E), jnp.int32),
        pltpu.VMEM((2, BLOCK_E), jnp.int32),
        pltpu.VMEM((BLOCK_E, D_HALF), jnp.float32),
        pltpu.VMEM((BLOCK_E, D_HALF), jnp.float32),
        pltpu.VMEM_SHARED((N_PAD, D_HALF), jnp.float32),
        pltpu.SemaphoreType.DMA,
        pltpu.SemaphoreType.DMA,
        pltpu.SemaphoreType.DMA,
        pltpu.SemaphoreType.DMA,
        pltpu.SemaphoreType.DMA,
        pltpu.SemaphoreType.DMA,
    ],
)(_sc_body)


def _scale_body(a_ref, ci_ref, o_ref):
    o_ref[:, :D_HALF] = a_ref[0] * ci_ref[...]
    o_ref[:, D_HALF:] = a_ref[1] * ci_ref[...]


_tc_scale = pl.pallas_call(
    _scale_body,
    grid=(N_NODES // _R,),
    in_specs=[
        pl.BlockSpec((N_CORES, _R, D_HALF), lambda i: (0, i, 0)),
        pl.BlockSpec((_R, 1), lambda i: (i, 0)),
    ],
    out_specs=pl.BlockSpec((_R, D_FEAT), lambda i: (i, 0)),
    out_shape=jax.ShapeDtypeStruct((N_NODES, D_FEAT), jnp.float32),
)


def kernel(src_feats, edge_index, cj, ci):
    src = edge_index[0].astype(jnp.int32)
    dst = edge_index[1].astype(jnp.int32)
    pad = E_PAD - N_EDGES
    src3 = jnp.concatenate([src, jnp.zeros((pad,), jnp.int32)])
    src3 = src3.reshape(N_TILES, N_BLOCKS, BLOCK_E)
    pad_dst = N_NODES + (jnp.arange(pad, dtype=jnp.int32) % (N_PAD - N_NODES))
    dst3 = jnp.concatenate([dst, pad_dst])
    dst3 = dst3.reshape(N_TILES, N_BLOCKS, BLOCK_E)
    idx = jnp.stack([src3, dst3], axis=2)

    w0, w1 = _tc_weight(src_feats, cj)
    zeros = jnp.zeros((N_PAD, D_HALF), jnp.float32)
    agg = _sc_gather_scatter(w0, w1, idx, zeros)
    return _tc_scale(agg.reshape(N_CORES, N_PAD, D_HALF), ci)


# speedup vs baseline: 1.2643x; 1.2609x over previous
import functools

import jax
import jax.numpy as jnp
from jax import lax
from jax.experimental import pallas as pl
from jax.experimental.pallas import tpu as pltpu
from jax.experimental.pallas import tpu_sc as plsc

N_NODES = 10000
N_EDGES = 160000
D_FEAT = 256
D_HALF = 128
N_CORES = 2
N_TILES = 16

BLOCK_E = 80
N_BLOCKS = 126
EDGES_PER_TILE = N_BLOCKS * BLOCK_E
E_PAD = EDGES_PER_TILE * N_TILES
N_PAD = 10240
ROWS_PER_TILE = N_PAD // N_TILES


def _weight_body(x_ref, cj_ref, w0_ref, w1_ref):
    w0_ref[...] = x_ref[:, :D_HALF] * cj_ref[...]
    w1_ref[...] = x_ref[:, D_HALF:] * cj_ref[...]


_R = 1000

_tc_weight = pl.pallas_call(
    _weight_body,
    grid=(N_NODES // _R,),
    in_specs=[
        pl.BlockSpec((_R, D_FEAT), lambda i: (i, 0)),
        pl.BlockSpec((_R, 1), lambda i: (i, 0)),
    ],
    out_specs=[
        pl.BlockSpec((_R, D_HALF), lambda i: (i, 0)),
        pl.BlockSpec((_R, D_HALF), lambda i: (i, 0)),
    ],
    out_shape=[
        jax.ShapeDtypeStruct((N_NODES, D_HALF), jnp.float32),
        jax.ShapeDtypeStruct((N_NODES, D_HALF), jnp.float32),
    ],
)


def _sc_body(w0_hbm, w1_hbm, idx_hbm, zero_hbm, out_hbm,
             ib0, ib1, rows0, rows1, acc,
             isem0, isem1, gsem0, gsem1, ssem0, ssem1):
    c = lax.axis_index("c")
    s = lax.axis_index("s")
    out_off = c * N_PAD

    def idx_start(i, ib, sem):
        pltpu.async_copy(idx_hbm.at[s, i], ib, sem)

    def idx_wait(ib, sem):
        pltpu.make_async_copy(idx_hbm.at[0, 0], ib, sem).wait()

    def start_gather(ib, rows, sem):
        @pl.when(c == 0)
        def _():
            pltpu.async_copy(w0_hbm.at[ib.at[0]], rows, sem)

        @pl.when(c == 1)
        def _():
            pltpu.async_copy(w1_hbm.at[ib.at[0]], rows, sem)

    def wait_gather(rows, sem):
        pltpu.make_async_copy(w0_hbm.at[pl.ds(0, BLOCK_E)], rows, sem).wait()

    def start_scatter(ib, rows, sem):
        pltpu.async_copy(rows, acc.at[ib.at[1]], sem, add=True)

    def wait_scatter(rows, sem):
        pltpu.make_async_copy(rows, acc.at[pl.ds(0, BLOCK_E)], sem).wait()

    idx_start(0, ib0, isem0)
    r0 = s * ROWS_PER_TILE
    pltpu.sync_copy(zero_hbm.at[pl.ds(r0, ROWS_PER_TILE)],
                    acc.at[pl.ds(r0, ROWS_PER_TILE)])
    plsc.subcore_barrier()

    idx_wait(ib0, isem0)
    start_gather(ib0, rows0, gsem0)

    def block2(k, carry):
        e = 2 * k
        wait_gather(rows0, gsem0)
        start_scatter(ib0, rows0, ssem0)

        @pl.when(k > 0)
        def _():
            wait_scatter(rows1, ssem1)

        idx_start(e + 1, ib1, isem1)
        idx_wait(ib1, isem1)
        start_gather(ib1, rows1, gsem1)
        wait_scatter(rows0, ssem0)

        @pl.when(e + 2 < N_BLOCKS)
        def _():
            idx_start(e + 2, ib0, isem0)
            idx_wait(ib0, isem0)

        wait_gather(rows1, gsem1)
        start_scatter(ib1, rows1, ssem1)

        @pl.when(e + 2 < N_BLOCKS)
        def _():
            start_gather(ib0, rows0, gsem0)

        return carry

    lax.fori_loop(0, N_BLOCKS // 2, block2, 0)

    wait_scatter(rows1, ssem1)
    plsc.subcore_barrier()
    pltpu.sync_copy(acc.at[pl.ds(r0, ROWS_PER_TILE)],
                    out_hbm.at[pl.ds(out_off + r0, ROWS_PER_TILE)])


_sc_gather_scatter = functools.partial(
    pl.kernel,
    out_type=jax.ShapeDtypeStruct((N_CORES * N_PAD, D_HALF), jnp.float32),
    mesh=plsc.VectorSubcoreMesh(core_axis_name="c", subcore_axis_name="s"),
    scratch_types=[
        pltpu.VMEM((2, BLOCK_E), jnp.int32),
        pltpu.VMEM((2, BLOCK_E), jnp.int32),
        pltpu.VMEM((BLOCK_E, D_HALF), jnp.float32),
        pltpu.VMEM((BLOCK_E, D_HALF), jnp.float32),
        pltpu.VMEM_SHARED((N_PAD, D_HALF), jnp.float32),
        pltpu.SemaphoreType.DMA,
        pltpu.SemaphoreType.DMA,
        pltpu.SemaphoreType.DMA,
        pltpu.SemaphoreType.DMA,
        pltpu.SemaphoreType.DMA,
        pltpu.SemaphoreType.DMA,
    ],
)(_sc_body)


def _scale_body(a_ref, ci_ref, o_ref):
    o_ref[:, :D_HALF] = a_ref[0] * ci_ref[...]
    o_ref[:, D_HALF:] = a_ref[1] * ci_ref[...]


_tc_scale = pl.pallas_call(
    _scale_body,
    grid=(N_NODES // _R,),
    in_specs=[
        pl.BlockSpec((N_CORES, _R, D_HALF), lambda i: (0, i, 0)),
        pl.BlockSpec((_R, 1), lambda i: (i, 0)),
    ],
    out_specs=pl.BlockSpec((_R, D_FEAT), lambda i: (i, 0)),
    out_shape=jax.ShapeDtypeStruct((N_NODES, D_FEAT), jnp.float32),
)


def kernel(src_feats, edge_index, cj, ci):
    src = edge_index[0].astype(jnp.int32)
    dst = edge_index[1].astype(jnp.int32)
    pad = E_PAD - N_EDGES
    src3 = jnp.concatenate([src, jnp.zeros((pad,), jnp.int32)])
    src3 = src3.reshape(N_TILES, N_BLOCKS, BLOCK_E)
    pad_dst = N_NODES + (jnp.arange(pad, dtype=jnp.int32) % (N_PAD - N_NODES))
    dst3 = jnp.concatenate([dst, pad_dst])
    dst3 = dst3.reshape(N_TILES, N_BLOCKS, BLOCK_E)
    idx = jnp.stack([src3, dst3], axis=2)

    w0, w1 = _tc_weight(src_feats, cj)
    zeros = jnp.zeros((N_PAD, D_HALF), jnp.float32)
    agg = _sc_gather_scatter(w0, w1, idx, zeros)
    return _tc_scale(agg.reshape(N_CORES, N_PAD, D_HALF), ci)


# speedup vs baseline: 1.3876x; 1.0975x over previous
import functools

import jax
import jax.numpy as jnp
from jax import lax
from jax.experimental import pallas as pl
from jax.experimental.pallas import tpu as pltpu
from jax.experimental.pallas import tpu_sc as plsc

N_NODES = 10000
N_EDGES = 160000
D_FEAT = 256
D_HALF = 128
N_CORES = 2
N_TILES = 16

BLOCK_E = 80
N_BLOCKS = 126
EDGES_PER_TILE = N_BLOCKS * BLOCK_E
E_PAD = EDGES_PER_TILE * N_TILES
N_PAD = 10240
ROWS_PER_TILE = N_PAD // N_TILES


def _weight_body(x_ref, cj_ref, w0_ref, w1_ref):
    w0_ref[...] = x_ref[:, :D_HALF] * cj_ref[...]
    w1_ref[...] = x_ref[:, D_HALF:] * cj_ref[...]


_R = 1000

_tc_weight = pl.pallas_call(
    _weight_body,
    grid=(N_NODES // _R,),
    in_specs=[
        pl.BlockSpec((_R, D_FEAT), lambda i: (i, 0)),
        pl.BlockSpec((_R, 1), lambda i: (i, 0)),
    ],
    out_specs=[
        pl.BlockSpec((_R, D_HALF), lambda i: (i, 0)),
        pl.BlockSpec((_R, D_HALF), lambda i: (i, 0)),
    ],
    out_shape=[
        jax.ShapeDtypeStruct((N_NODES, D_HALF), jnp.float32),
        jax.ShapeDtypeStruct((N_NODES, D_HALF), jnp.float32),
    ],
)


def _sc_body(w0_hbm, w1_hbm, idx_hbm, zero_hbm, out_hbm,
             ib0, ib1, rows0, rows1, acc,
             isem0, isem1, gsem0, gsem1, ssem0, ssem1):
    c = lax.axis_index("c")
    s = lax.axis_index("s")
    out_off = c * N_PAD

    def idx_start(i, ib, sem):
        pltpu.async_copy(idx_hbm.at[s, i], ib, sem)

    def idx_wait(ib, sem):
        pltpu.make_async_copy(idx_hbm.at[0, 0], ib, sem).wait()

    def start_gather(ib, rows, sem):
        @pl.when(c == 0)
        def _():
            pltpu.async_copy(w0_hbm.at[ib.at[0]], rows, sem)

        @pl.when(c == 1)
        def _():
            pltpu.async_copy(w1_hbm.at[ib.at[0]], rows, sem)

    def wait_gather(rows, sem):
        pltpu.make_async_copy(w0_hbm.at[pl.ds(0, BLOCK_E)], rows, sem).wait()

    def scatter(ib, rows):
        pltpu.sync_copy(rows, acc.at[ib.at[1]], add=True)

    idx_start(0, ib0, isem0)
    idx_start(1, ib1, isem1)
    r0 = s * ROWS_PER_TILE
    pltpu.sync_copy(zero_hbm.at[pl.ds(r0, ROWS_PER_TILE)],
                    acc.at[pl.ds(r0, ROWS_PER_TILE)])
    plsc.subcore_barrier()

    idx_wait(ib0, isem0)
    start_gather(ib0, rows0, gsem0)

    def block2(k, carry):
        i = 2 * k
        idx_wait(ib1, isem1)
        wait_gather(rows0, gsem0)
        start_gather(ib1, rows1, gsem1)
        scatter(ib0, rows0)

        @pl.when(i + 2 < N_BLOCKS)
        def _():
            idx_start(i + 2, ib0, isem0)
            idx_wait(ib0, isem0)

        wait_gather(rows1, gsem1)

        @pl.when(i + 2 < N_BLOCKS)
        def _():
            start_gather(ib0, rows0, gsem0)

        scatter(ib1, rows1)

        @pl.when(i + 3 < N_BLOCKS)
        def _():
            idx_start(i + 3, ib1, isem1)

        return carry

    lax.fori_loop(0, N_BLOCKS // 2, block2, 0)

    plsc.subcore_barrier()
    pltpu.sync_copy(acc.at[pl.ds(r0, ROWS_PER_TILE)],
                    out_hbm.at[pl.ds(out_off + r0, ROWS_PER_TILE)])


_sc_gather_scatter = functools.partial(
    pl.kernel,
    out_type=jax.ShapeDtypeStruct((N_CORES * N_PAD, D_HALF), jnp.float32),
    mesh=plsc.VectorSubcoreMesh(core_axis_name="c", subcore_axis_name="s"),
    scratch_types=[
        pltpu.VMEM((2, BLOCK_E), jnp.int32),
        pltpu.VMEM((2, BLOCK_E), jnp.int32),
        pltpu.VMEM((BLOCK_E, D_HALF), jnp.float32),
        pltpu.VMEM((BLOCK_E, D_HALF), jnp.float32),
        pltpu.VMEM_SHARED((N_PAD, D_HALF), jnp.float32),
        pltpu.SemaphoreType.DMA,
        pltpu.SemaphoreType.DMA,
        pltpu.SemaphoreType.DMA,
        pltpu.SemaphoreType.DMA,
        pltpu.SemaphoreType.DMA,
        pltpu.SemaphoreType.DMA,
    ],
)(_sc_body)


def _scale_body(a_ref, ci_ref, o_ref):
    o_ref[:, :D_HALF] = a_ref[0] * ci_ref[...]
    o_ref[:, D_HALF:] = a_ref[1] * ci_ref[...]


_tc_scale = pl.pallas_call(
    _scale_body,
    grid=(N_NODES // _R,),
    in_specs=[
        pl.BlockSpec((N_CORES, _R, D_HALF), lambda i: (0, i, 0)),
        pl.BlockSpec((_R, 1), lambda i: (i, 0)),
    ],
    out_specs=pl.BlockSpec((_R, D_FEAT), lambda i: (i, 0)),
    out_shape=jax.ShapeDtypeStruct((N_NODES, D_FEAT), jnp.float32),
)


def kernel(src_feats, edge_index, cj, ci):
    src = edge_index[0].astype(jnp.int32)
    dst = edge_index[1].astype(jnp.int32)
    pad = E_PAD - N_EDGES
    src3 = jnp.concatenate([src, jnp.zeros((pad,), jnp.int32)])
    src3 = src3.reshape(N_TILES, N_BLOCKS, BLOCK_E)
    pad_dst = N_NODES + (jnp.arange(pad, dtype=jnp.int32) % (N_PAD - N_NODES))
    dst3 = jnp.concatenate([dst, pad_dst])
    dst3 = dst3.reshape(N_TILES, N_BLOCKS, BLOCK_E)
    idx = jnp.stack([src3, dst3], axis=2)

    w0, w1 = _tc_weight(src_feats, cj)
    zeros = jnp.zeros((N_PAD, D_HALF), jnp.float32)
    agg = _sc_gather_scatter(w0, w1, idx, zeros)
    return _tc_scale(agg.reshape(N_CORES, N_PAD, D_HALF), ci)


# speedup vs baseline: 1.6926x; 1.2198x over previous
import functools

import jax
import jax.numpy as jnp
from jax import lax
from jax.experimental import pallas as pl
from jax.experimental.pallas import tpu as pltpu
from jax.experimental.pallas import tpu_sc as plsc

N_NODES = 10000
N_EDGES = 160000
D_FEAT = 256
D_HALF = 128
N_CORES = 2
N_TILES = 16

BLOCK_E = 80
N_BLOCKS = 126
EDGES_PER_TILE = N_BLOCKS * BLOCK_E
E_PAD = EDGES_PER_TILE * N_TILES
N_PAD = 10240
ROWS_PER_TILE = N_PAD // N_TILES


def _weight_body(x_ref, cj_ref, w0_ref, w1_ref):
    w0_ref[...] = x_ref[:, :D_HALF] * cj_ref[...]
    w1_ref[...] = x_ref[:, D_HALF:] * cj_ref[...]


_R = 1000

_tc_weight = pl.pallas_call(
    _weight_body,
    grid=(N_NODES // _R,),
    in_specs=[
        pl.BlockSpec((_R, D_FEAT), lambda i: (i, 0)),
        pl.BlockSpec((_R, 1), lambda i: (i, 0)),
    ],
    out_specs=[
        pl.BlockSpec((_R, D_HALF), lambda i: (i, 0)),
        pl.BlockSpec((_R, D_HALF), lambda i: (i, 0)),
    ],
    out_shape=[
        jax.ShapeDtypeStruct((N_NODES, D_HALF), jnp.float32),
        jax.ShapeDtypeStruct((N_NODES, D_HALF), jnp.float32),
    ],
)


N_BUF = 3
N_IBUF = 2 * N_BUF


def _sc_body(w0_hbm, w1_hbm, idx_hbm, zero_hbm, out_hbm,
             ibs, rowss, acc, isems, gsems):
    c = lax.axis_index("c")
    s = lax.axis_index("s")
    out_off = c * N_PAD

    def idx_start(i, ji):
        pltpu.async_copy(idx_hbm.at[s, i], ibs[ji], isems[ji])

    def idx_wait(ji):
        pltpu.make_async_copy(idx_hbm.at[0, 0], ibs[ji], isems[ji]).wait()

    def start_gather(ji, jr):
        @pl.when(c == 0)
        def _():
            pltpu.async_copy(w0_hbm.at[ibs[ji].at[0]], rowss[jr], gsems[jr])

        @pl.when(c == 1)
        def _():
            pltpu.async_copy(w1_hbm.at[ibs[ji].at[0]], rowss[jr], gsems[jr])

    def wait_gather(jr):
        pltpu.make_async_copy(w0_hbm.at[pl.ds(0, BLOCK_E)], rowss[jr],
                              gsems[jr]).wait()

    def scatter(ji, jr):
        pltpu.sync_copy(rowss[jr], acc.at[ibs[ji].at[1]], add=True)

    for j in range(N_IBUF):
        idx_start(j, j)
    r0 = s * ROWS_PER_TILE
    pltpu.sync_copy(zero_hbm.at[pl.ds(r0, ROWS_PER_TILE)],
                    acc.at[pl.ds(r0, ROWS_PER_TILE)])
    plsc.subcore_barrier()

    for j in range(N_BUF):
        idx_wait(j)
        start_gather(j, j)

    def ring(k, carry):
        base = N_IBUF * k
        for j in range(N_IBUF):
            i = base + j
            jr = j % N_BUF
            wait_gather(jr)
            scatter(j, jr)

            @pl.when(i + N_IBUF < N_BLOCKS)
            def _():
                idx_start(i + N_IBUF, j)

            @pl.when(i + N_BUF < N_BLOCKS)
            def _():
                idx_wait((j + N_BUF) % N_IBUF)
                start_gather((j + N_BUF) % N_IBUF, jr)

        return carry

    lax.fori_loop(0, N_BLOCKS // N_IBUF, ring, 0)

    plsc.subcore_barrier()
    pltpu.sync_copy(acc.at[pl.ds(r0, ROWS_PER_TILE)],
                    out_hbm.at[pl.ds(out_off + r0, ROWS_PER_TILE)])


_sc_gather_scatter = functools.partial(
    pl.kernel,
    out_type=jax.ShapeDtypeStruct((N_CORES * N_PAD, D_HALF), jnp.float32),
    mesh=plsc.VectorSubcoreMesh(core_axis_name="c", subcore_axis_name="s"),
    scratch_types=[
        [pltpu.VMEM((2, BLOCK_E), jnp.int32) for _ in range(N_IBUF)],
        [pltpu.VMEM((BLOCK_E, D_HALF), jnp.float32) for _ in range(N_BUF)],
        pltpu.VMEM_SHARED((N_PAD, D_HALF), jnp.float32),
        [pltpu.SemaphoreType.DMA for _ in range(N_IBUF)],
        [pltpu.SemaphoreType.DMA for _ in range(N_BUF)],
    ],
)(_sc_body)


def _scale_body(a_ref, ci_ref, o_ref):
    o_ref[:, :D_HALF] = a_ref[0] * ci_ref[...]
    o_ref[:, D_HALF:] = a_ref[1] * ci_ref[...]


_tc_scale = pl.pallas_call(
    _scale_body,
    grid=(N_NODES // _R,),
    in_specs=[
        pl.BlockSpec((N_CORES, _R, D_HALF), lambda i: (0, i, 0)),
        pl.BlockSpec((_R, 1), lambda i: (i, 0)),
    ],
    out_specs=pl.BlockSpec((_R, D_FEAT), lambda i: (i, 0)),
    out_shape=jax.ShapeDtypeStruct((N_NODES, D_FEAT), jnp.float32),
)


def kernel(src_feats, edge_index, cj, ci):
    src = edge_index[0].astype(jnp.int32)
    dst = edge_index[1].astype(jnp.int32)
    pad = E_PAD - N_EDGES
    src3 = jnp.concatenate([src, jnp.zeros((pad,), jnp.int32)])
    src3 = src3.reshape(N_TILES, N_BLOCKS, BLOCK_E)
    pad_dst = N_NODES + (jnp.arange(pad, dtype=jnp.int32) % (N_PAD - N_NODES))
    dst3 = jnp.concatenate([dst, pad_dst])
    dst3 = dst3.reshape(N_TILES, N_BLOCKS, BLOCK_E)
    idx = jnp.stack([src3, dst3], axis=2)

    w0, w1 = _tc_weight(src_feats, cj)
    zeros = jnp.zeros((N_PAD, D_HALF), jnp.float32)
    agg = _sc_gather_scatter(w0, w1, idx, zeros)
    return _tc_scale(agg.reshape(N_CORES, N_PAD, D_HALF), ci)
